# Initial kernel scaffold; baseline (speedup 1.0000x reference)
import jax
import jax.numpy as jnp
from jax.experimental import pallas as pl


def kernel(x, batch, pos_embedding):
    raise NotImplementedError("write your pallas kernel here")


# speedup vs baseline: 1.7047x; 1.7047x over previous
import functools

import jax
import jax.numpy as jnp
from jax import lax
from jax.experimental import pallas as pl
from jax.experimental.pallas import tpu as pltpu
from jax.experimental.pallas import tpu_sc as plsc

N = 50000
HIDDEN = 250
MAX_NODES = 1000
NUM_GRAPHS = 100

NC = 2
NS = 16
L = 16
NW = NC * NS

C = 80
NCHUNK = N // C
CHUNKS_PER_TILE = (NCHUNK + NW - 1) // NW
G_PAD = 112
ROW_VECS = HIDDEN // L
TAIL_OFF = HIDDEN - L
H_PAD = 256


def _sc_body(x_hbm, batch_hbm, table_hbm, out_hbm,
             batch_full, x_v, rows_v, out_v, b_v, idx_v, starts_v, sem):
    wid = lax.axis_index("s") * NC + lax.axis_index("c")

    pltpu.sync_copy(batch_hbm, batch_full)
    iota = lax.iota(jnp.int32, L)
    for grp in range(G_PAD // L):
        g = iota + grp * L

        def bs_step(_, carry):
            lo, hi = carry
            mid = lax.div(lo + hi, 2)
            probe = plsc.load_gather(batch_full, [jnp.minimum(mid, N - 1)])
            go = lo < hi
            pred = jnp.logical_and(probe < g, go)
            lo2 = jnp.where(pred, mid + 1, lo)
            hi2 = jnp.where(jnp.logical_and(jnp.logical_not(probe < g), go),
                            mid, hi)
            return lo2, hi2

        lo0 = jnp.zeros((L,), jnp.int32)
        hi0 = jnp.full((L,), N, jnp.int32)
        lo, hi = lax.fori_loop(0, 16, bs_step, (lo0, hi0))
        starts_v[pl.ds(grp * L, L)] = lo

    def chunk_body(k, _):
        chunk = wid + k * NW

        @pl.when(chunk < NCHUNK)
        def _():
            a = chunk * C
            pltpu.sync_copy(batch_hbm.at[pl.ds(a, C)], b_v)
            pltpu.sync_copy(x_hbm.at[pl.ds(a, C), :], x_v)
            for j in range(C // L):
                b = b_v[pl.ds(j * L, L)]
                s = plsc.load_gather(starts_v, [b])
                pos = (a + j * L) + iota - s
                pos = jnp.minimum(jnp.maximum(pos, 0), MAX_NODES - 1)
                idx_v[pl.ds(j * L, L)] = pos
            pltpu.async_copy(table_hbm.at[idx_v], rows_v, sem).wait()

            def row_body(r, _):
                for c in range(ROW_VECS):
                    w = pl.ds(c * L, L)
                    out_v[r, w] = x_v[r, w] + rows_v[r, w]
                w = pl.ds(TAIL_OFF, L)
                out_v[r, w] = x_v[r, w] + rows_v[r, w]
                return 0

            lax.fori_loop(0, C, row_body, 0)
            pltpu.sync_copy(out_v, out_hbm.at[pl.ds(a, C), :])

        return 0

    lax.fori_loop(0, CHUNKS_PER_TILE, chunk_body, 0)


@jax.jit
def kernel(x, batch, pos_embedding):
    mesh = plsc.VectorSubcoreMesh(core_axis_name="c", subcore_axis_name="s",
                                  num_cores=NC, num_subcores=NS)
    f = pl.kernel(
        _sc_body,
        out_type=jax.ShapeDtypeStruct((N, HIDDEN), jnp.float32),
        mesh=mesh,
        scratch_types=[
            pltpu.VMEM((N,), jnp.int32),
            pltpu.VMEM((C, HIDDEN), jnp.float32),
            pltpu.VMEM((C, H_PAD), jnp.float32),
            pltpu.VMEM((C, HIDDEN), jnp.float32),
            pltpu.VMEM((C,), jnp.int32),
            pltpu.VMEM((C,), jnp.int32),
            pltpu.VMEM((G_PAD,), jnp.int32),
            pltpu.SemaphoreType.DMA,
        ],
        compiler_params=pltpu.CompilerParams(needs_layout_passes=False,
                                             use_tc_tiling_on_sc=False),
    )
    table = jnp.pad(pos_embedding, ((0, 0), (0, H_PAD - HIDDEN)))
    return f(x, batch, table)


# speedup vs baseline: 1.9207x; 1.1267x over previous
import jax
import jax.numpy as jnp
from jax import lax
from jax.experimental import pallas as pl
from jax.experimental.pallas import tpu as pltpu
from jax.experimental.pallas import tpu_sc as plsc

N = 50000
HIDDEN = 250
MAX_NODES = 1000
NUM_GRAPHS = 100

NC = 2
NS = 16
L = 16
NW = NC * NS

C = 80
NCHUNK = N // C
K_ITERS = (NCHUNK + NW - 1) // NW
G_PAD = 112
ROW_VECS = HIDDEN // L
TAIL_OFF = HIDDEN - L
H_PAD = 256


def _sc_body(x_hbm, batch_hbm, table_hbm, out_hbm, starts_v,
             bs0, bs1, xs0, xs1, gs0, gs1, os0, os1):
    wid = lax.axis_index("s") * NC + lax.axis_index("c")
    iota = lax.iota(jnp.int32, L)

    def phase1(batch_full):
        pltpu.sync_copy(batch_hbm, batch_full)
        for grp in range(G_PAD // L):
            g = iota + grp * L

            def bs_step(_, carry):
                lo, hi = carry
                mid = lax.div(lo + hi, 2)
                probe = plsc.load_gather(batch_full, [jnp.minimum(mid, N - 1)])
                go = lo < hi
                lt = probe < g
                lo2 = jnp.where(jnp.logical_and(lt, go), mid + 1, lo)
                hi2 = jnp.where(jnp.logical_and(jnp.logical_not(lt), go),
                                mid, hi)
                return lo2, hi2

            lo0 = jnp.zeros((L,), jnp.int32)
            hi0 = jnp.full((L,), N, jnp.int32)
            lo, _ = lax.fori_loop(0, 16, bs_step, (lo0, hi0))
            starts_v[pl.ds(grp * L, L)] = lo

    pl.run_scoped(phase1, pltpu.VMEM((N,), jnp.int32))

    def phase2(x0, x1, r0, r1, o0, o1, b0, b1, i0, i1):
        X, R, O, B, I = [x0, x1], [r0, r1], [o0, o1], [b0, b1], [i0, i1]
        BS, XS, GS, OS = [bs0, bs1], [xs0, xs1], [gs0, gs1], [os0, os1]

        def issue(k):
            s = k & 1
            a = (wid + k * NW) * C
            pltpu.async_copy(batch_hbm.at[pl.ds(a, C)], B[s], BS[s])
            pltpu.async_copy(x_hbm.at[pl.ds(a, C), :], X[s], XS[s])

        @pl.when(wid < NCHUNK)
        def _():
            issue(0)

        for k in range(K_ITERS):
            s = k & 1
            chunk = wid + k * NW

            @pl.when(chunk < NCHUNK)
            def _(k=k, s=s, chunk=chunk):
                a = chunk * C
                if k + 1 < K_ITERS:
                    @pl.when(chunk + NW < NCHUNK)
                    def _():
                        issue(k + 1)
                pltpu.make_async_copy(
                    batch_hbm.at[pl.ds(a, C)], B[s], BS[s]).wait()
                for j in range(C // L):
                    b = B[s][pl.ds(j * L, L)]
                    st = plsc.load_gather(starts_v, [b])
                    pos = (a + j * L) + iota - st
                    pos = jnp.minimum(jnp.maximum(pos, 0), MAX_NODES - 1)
                    I[s][pl.ds(j * L, L)] = pos
                pltpu.async_copy(table_hbm.at[I[s]], R[s], GS[s])
                pltpu.make_async_copy(
                    x_hbm.at[pl.ds(a, C), :], X[s], XS[s]).wait()
                pltpu.make_async_copy(table_hbm.at[I[s]], R[s], GS[s]).wait()
                if k >= 2:
                    pltpu.make_async_copy(
                        O[s], out_hbm.at[pl.ds(a - 2 * NW * C, C), :],
                        OS[s]).wait()

                def row_body(r, _):
                    for c in range(ROW_VECS):
                        w = pl.ds(c * L, L)
                        O[s][r, w] = X[s][r, w] + R[s][r, w]
                    w = pl.ds(TAIL_OFF, L)
                    O[s][r, w] = X[s][r, w] + R[s][r, w]
                    return 0

                lax.fori_loop(0, C, row_body, 0)
                pltpu.async_copy(O[s], out_hbm.at[pl.ds(a, C), :], OS[s])

        for k in (K_ITERS - 2, K_ITERS - 1):
            s = k & 1
            chunk = wid + k * NW

            @pl.when(chunk < NCHUNK)
            def _(s=s, chunk=chunk):
                pltpu.make_async_copy(
                    O[s], out_hbm.at[pl.ds(chunk * C, C), :], OS[s]).wait()

    pl.run_scoped(
        phase2,
        pltpu.VMEM((C, HIDDEN), jnp.float32),
        pltpu.VMEM((C, HIDDEN), jnp.float32),
        pltpu.VMEM((C, H_PAD), jnp.float32),
        pltpu.VMEM((C, H_PAD), jnp.float32),
        pltpu.VMEM((C, HIDDEN), jnp.float32),
        pltpu.VMEM((C, HIDDEN), jnp.float32),
        pltpu.VMEM((C,), jnp.int32),
        pltpu.VMEM((C,), jnp.int32),
        pltpu.VMEM((C,), jnp.int32),
        pltpu.VMEM((C,), jnp.int32),
    )


@jax.jit
def kernel(x, batch, pos_embedding):
    mesh = plsc.VectorSubcoreMesh(core_axis_name="c", subcore_axis_name="s",
                                  num_cores=NC, num_subcores=NS)
    f = pl.kernel(
        _sc_body,
        out_type=jax.ShapeDtypeStruct((N, HIDDEN), jnp.float32),
        mesh=mesh,
        scratch_types=[
            pltpu.VMEM((G_PAD,), jnp.int32),
            pltpu.SemaphoreType.DMA,
            pltpu.SemaphoreType.DMA,
            pltpu.SemaphoreType.DMA,
            pltpu.SemaphoreType.DMA,
            pltpu.SemaphoreType.DMA,
            pltpu.SemaphoreType.DMA,
            pltpu.SemaphoreType.DMA,
            pltpu.SemaphoreType.DMA,
        ],
        compiler_params=pltpu.CompilerParams(needs_layout_passes=False,
                                             use_tc_tiling_on_sc=False),
    )
    table = jnp.pad(pos_embedding, ((0, 0), (0, H_PAD - HIDDEN)))
    return f(x, batch, table)


# speedup vs baseline: 2.0255x; 1.0546x over previous
import jax
import jax.numpy as jnp
from jax import lax
from jax.experimental import pallas as pl
from jax.experimental.pallas import tpu as pltpu
from jax.experimental.pallas import tpu_sc as plsc

N = 50000
HIDDEN = 250
MAX_NODES = 1000
NUM_GRAPHS = 100

NC = 2
NS = 16
L = 16
NW = NC * NS

C = 80
NCHUNK = N // C
K_ITERS = (NCHUNK + NW - 1) // NW
G_PAD = 112
ROW_VECS = HIDDEN // L
TAIL_OFF = HIDDEN - L
H_PAD = 256


def _sc_body(x_hbm, batch_hbm, table_hbm, out_hbm, starts_v,
             bs0, bs1, xs0, xs1, gs0, gs1, os0, os1):
    wid = lax.axis_index("s") * NC + lax.axis_index("c")
    iota = lax.iota(jnp.int32, L)

    def phase1(batch_full):
        pltpu.sync_copy(batch_hbm, batch_full)
        for grp in range(G_PAD // L):
            g = iota + grp * L

            def bs_step(_, carry):
                lo, hi = carry
                mid = lax.div(lo + hi, 2)
                probe = plsc.load_gather(batch_full, [jnp.minimum(mid, N - 1)])
                go = lo < hi
                lt = probe < g
                lo2 = jnp.where(jnp.logical_and(lt, go), mid + 1, lo)
                hi2 = jnp.where(jnp.logical_and(jnp.logical_not(lt), go),
                                mid, hi)
                return lo2, hi2

            lo0 = jnp.zeros((L,), jnp.int32)
            hi0 = jnp.full((L,), N, jnp.int32)
            lo, _ = lax.fori_loop(0, 16, bs_step, (lo0, hi0))
            starts_v[pl.ds(grp * L, L)] = lo

    pl.run_scoped(phase1, pltpu.VMEM((N,), jnp.int32))

    def phase2(x0, x1, r0, r1, o0, o1, b0, b1, i0, i1):
        X, R, O, B, I = [x0, x1], [r0, r1], [o0, o1], [b0, b1], [i0, i1]
        BS, XS, GS, OS = [bs0, bs1], [xs0, xs1], [gs0, gs1], [os0, os1]

        def issue_b(k):
            s = k & 1
            a = (wid + k * NW) * C
            pltpu.async_copy(batch_hbm.at[pl.ds(a, C)], B[s], BS[s])

        def issue_x(k):
            s = k & 1
            a = (wid + k * NW) * C
            pltpu.async_copy(x_hbm.at[pl.ds(a, C), :], X[s], XS[s])

        def stage_a(k):
            s = k & 1
            a = (wid + k * NW) * C
            pltpu.make_async_copy(batch_hbm.at[pl.ds(a, C)], B[s],
                                  BS[s]).wait()
            for j in range(C // L):
                b = B[s][pl.ds(j * L, L)]
                st = plsc.load_gather(starts_v, [b])
                pos = (a + j * L) + iota - st
                pos = jnp.minimum(jnp.maximum(pos, 0), MAX_NODES - 1)
                I[s][pl.ds(j * L, L)] = pos
            pltpu.async_copy(table_hbm.at[I[s]], R[s], GS[s])

        def stage_b(k):
            s = k & 1
            a = (wid + k * NW) * C
            pltpu.make_async_copy(x_hbm.at[pl.ds(a, C), :], X[s],
                                  XS[s]).wait()
            pltpu.make_async_copy(table_hbm.at[I[s]], R[s], GS[s]).wait()
            if k >= 2:
                pltpu.make_async_copy(
                    O[s], out_hbm.at[pl.ds(a - 2 * NW * C, C), :],
                    OS[s]).wait()

            @plsc.parallel_loop(0, C, step=1, unroll=2)
            def _(r):
                for c in range(ROW_VECS):
                    w = pl.ds(c * L, L)
                    O[s][r, w] = X[s][r, w] + R[s][r, w]
                w = pl.ds(TAIL_OFF, L)
                O[s][r, w] = X[s][r, w] + R[s][r, w]

            pltpu.async_copy(O[s], out_hbm.at[pl.ds(a, C), :], OS[s])

        def guarded(k, fn):
            if 0 <= k < K_ITERS:
                @pl.when(wid + k * NW < NCHUNK)
                def _():
                    fn(k)

        guarded(0, issue_b)
        guarded(0, issue_x)
        guarded(1, issue_b)
        for k in range(K_ITERS + 1):
            guarded(k, stage_a)
            guarded(k - 1, stage_b)
            guarded(k + 1, issue_x)
            guarded(k + 2, issue_b)

        def drain_out(k):
            s = k & 1
            a = (wid + k * NW) * C
            pltpu.make_async_copy(O[s], out_hbm.at[pl.ds(a, C), :],
                                  OS[s]).wait()

        guarded(K_ITERS - 2, drain_out)
        guarded(K_ITERS - 1, drain_out)

    pl.run_scoped(
        phase2,
        pltpu.VMEM((C, HIDDEN), jnp.float32),
        pltpu.VMEM((C, HIDDEN), jnp.float32),
        pltpu.VMEM((C, H_PAD), jnp.float32),
        pltpu.VMEM((C, H_PAD), jnp.float32),
        pltpu.VMEM((C, HIDDEN), jnp.float32),
        pltpu.VMEM((C, HIDDEN), jnp.float32),
        pltpu.VMEM((C,), jnp.int32),
        pltpu.VMEM((C,), jnp.int32),
        pltpu.VMEM((C,), jnp.int32),
        pltpu.VMEM((C,), jnp.int32),
    )


@jax.jit
def kernel(x, batch, pos_embedding):
    mesh = plsc.VectorSubcoreMesh(core_axis_name="c", subcore_axis_name="s",
                                  num_cores=NC, num_subcores=NS)
    f = pl.kernel(
        _sc_body,
        out_type=jax.ShapeDtypeStruct((N, HIDDEN), jnp.float32),
        mesh=mesh,
        scratch_types=[
            pltpu.VMEM((G_PAD,), jnp.int32),
            pltpu.SemaphoreType.DMA,
            pltpu.SemaphoreType.DMA,
            pltpu.SemaphoreType.DMA,
            pltpu.SemaphoreType.DMA,
            pltpu.SemaphoreType.DMA,
            pltpu.SemaphoreType.DMA,
            pltpu.SemaphoreType.DMA,
            pltpu.SemaphoreType.DMA,
        ],
        compiler_params=pltpu.CompilerParams(needs_layout_passes=False,
                                             use_tc_tiling_on_sc=False),
    )
    table = jnp.pad(pos_embedding, ((0, 0), (0, H_PAD - HIDDEN)))
    return f(x, batch, table)


# speedup vs baseline: 4.5700x; 2.2563x over previous
import jax
import jax.numpy as jnp
from jax import lax
from jax.experimental import pallas as pl
from jax.experimental.pallas import tpu as pltpu
from jax.experimental.pallas import tpu_sc as plsc

N = 50000
HIDDEN = 250
MAX_NODES = 1000
NUM_GRAPHS = 100

NC = 2
NS = 16
L = 16
NW = NC * NS

C = 80
NCHUNK = N // C
K_ITERS = (NCHUNK + NW - 1) // NW
G_PAD = 112
ROW_VECS = HIDDEN // L
TAIL_OFF = HIDDEN - L
H_PAD = 256


def _sc_body(x_hbm, batch_hbm, table_hbm, out_hbm, starts_v,
             bs0, bs1, xs0, xs1, gs0, gs1, os0, os1):
    wid = lax.axis_index("s") * NC + lax.axis_index("c")
    iota = lax.iota(jnp.int32, L)

    def phase1(batch_full):
        pltpu.sync_copy(batch_hbm, batch_full)
        for grp in range(G_PAD // L):
            g = iota + grp * L

            def bs_step(_, carry):
                lo, hi = carry
                mid = lax.div(lo + hi, 2)
                probe = plsc.load_gather(batch_full, [jnp.minimum(mid, N - 1)])
                go = lo < hi
                lt = probe < g
                lo2 = jnp.where(jnp.logical_and(lt, go), mid + 1, lo)
                hi2 = jnp.where(jnp.logical_and(jnp.logical_not(lt), go),
                                mid, hi)
                return lo2, hi2

            lo0 = jnp.zeros((L,), jnp.int32)
            hi0 = jnp.full((L,), N, jnp.int32)
            lo, _ = lax.fori_loop(0, 16, bs_step, (lo0, hi0))
            starts_v[pl.ds(grp * L, L)] = lo

    pl.run_scoped(phase1, pltpu.VMEM((N,), jnp.int32))

    def phase2(x0, x1, r0, r1, o0, o1, b0, b1, i0, i1):
        X, R, O, B, I = [x0, x1], [r0, r1], [o0, o1], [b0, b1], [i0, i1]
        BS, XS, GS, OS = [bs0, bs1], [xs0, xs1], [gs0, gs1], [os0, os1]

        def issue_b(k):
            s = k & 1
            a = (wid + k * NW) * C
            pltpu.async_copy(batch_hbm.at[pl.ds(a, C)], B[s], BS[s])

        def issue_x(k):
            s = k & 1
            a = (wid + k * NW) * C
            pltpu.async_copy(x_hbm.at[pl.ds(a, C), :], X[s], XS[s])

        def stage_a(k):
            s = k & 1
            a = (wid + k * NW) * C
            pltpu.make_async_copy(batch_hbm.at[pl.ds(a, C)], B[s],
                                  BS[s]).wait()
            for j in range(C // L):
                b = B[s][pl.ds(j * L, L)]
                st = plsc.load_gather(starts_v, [b])
                pos = (a + j * L) + iota - st
                pos = jnp.minimum(jnp.maximum(pos, 0), MAX_NODES - 1)
                I[s][pl.ds(j * L, L)] = pos
            pltpu.async_copy(table_hbm.at[I[s]], R[s], GS[s])

        def stage_b(k):
            s = k & 1
            a = (wid + k * NW) * C
            pltpu.make_async_copy(x_hbm.at[pl.ds(a, C), :], X[s],
                                  XS[s]).wait()
            pltpu.make_async_copy(table_hbm.at[I[s]], R[s], GS[s]).wait()
            if k >= 2:
                pltpu.make_async_copy(
                    O[s], out_hbm.at[pl.ds(a - 2 * NW * C, C), :],
                    OS[s]).wait()

            @plsc.parallel_loop(0, C, step=1, unroll=2)
            def _(r):
                for c in range(ROW_VECS):
                    w = pl.ds(c * L, L)
                    O[s][r, w] = X[s][r, w] + R[s][r, w]
                w = pl.ds(TAIL_OFF, L)
                O[s][r, w] = X[s][r, w] + R[s][r, w]

            pltpu.async_copy(O[s], out_hbm.at[pl.ds(a, C), :], OS[s])

        def guarded(k, fn):
            if 0 <= k < K_ITERS:
                @pl.when(wid + k * NW < NCHUNK)
                def _():
                    fn(k)

        guarded(0, issue_b)
        guarded(0, issue_x)
        guarded(1, issue_b)
        for k in range(K_ITERS + 1):
            guarded(k, stage_a)
            guarded(k - 1, stage_b)
            guarded(k + 1, issue_x)
            guarded(k + 2, issue_b)

        def drain_out(k):
            s = k & 1
            a = (wid + k * NW) * C
            pltpu.make_async_copy(O[s], out_hbm.at[pl.ds(a, C), :],
                                  OS[s]).wait()

        guarded(K_ITERS - 2, drain_out)
        guarded(K_ITERS - 1, drain_out)

    pl.run_scoped(
        phase2,
        pltpu.VMEM((C, HIDDEN), jnp.float32),
        pltpu.VMEM((C, HIDDEN), jnp.float32),
        pltpu.VMEM((C, H_PAD), jnp.float32),
        pltpu.VMEM((C, H_PAD), jnp.float32),
        pltpu.VMEM((C, HIDDEN), jnp.float32),
        pltpu.VMEM((C, HIDDEN), jnp.float32),
        pltpu.VMEM((C,), jnp.int32),
        pltpu.VMEM((C,), jnp.int32),
        pltpu.VMEM((C,), jnp.int32),
        pltpu.VMEM((C,), jnp.int32),
    )


@jax.jit
def kernel(x, batch, pos_embedding):
    mesh = plsc.VectorSubcoreMesh(core_axis_name="c", subcore_axis_name="s",
                                  num_cores=NC, num_subcores=NS)
    f = pl.kernel(
        _sc_body,
        out_type=jax.ShapeDtypeStruct((N, HIDDEN), jnp.float32),
        mesh=mesh,
        scratch_types=[
            pltpu.VMEM((G_PAD,), jnp.int32),
            pltpu.SemaphoreType.DMA,
            pltpu.SemaphoreType.DMA,
            pltpu.SemaphoreType.DMA,
            pltpu.SemaphoreType.DMA,
            pltpu.SemaphoreType.DMA,
            pltpu.SemaphoreType.DMA,
            pltpu.SemaphoreType.DMA,
            pltpu.SemaphoreType.DMA,
        ],
        compiler_params=pltpu.CompilerParams(needs_layout_passes=False,
                                             use_tc_tiling_on_sc=True),
    )
    table = jnp.pad(pos_embedding, ((0, 0), (0, H_PAD - HIDDEN)))
    return f(x, batch, table)


# speedup vs baseline: 4.5937x; 1.0052x over previous
import jax
import jax.numpy as jnp
from jax import lax
from jax.experimental import pallas as pl
from jax.experimental.pallas import tpu as pltpu
from jax.experimental.pallas import tpu_sc as plsc

N = 50000
HIDDEN = 250
MAX_NODES = 1000
NUM_GRAPHS = 100

NC = 2
NS = 16
L = 16
NW = NC * NS

C = 80
NCHUNK = N // C
K_ITERS = (NCHUNK + NW - 1) // NW
G_PAD = 112
ROW_VECS = HIDDEN // L
TAIL_OFF = HIDDEN - L
H_PAD = 256


def _sc_body(x_hbm, batch_hbm, table_hbm, out_hbm, starts_v,
             bs0, bs1, xs0, xs1, xs2, gs0, gs1, os0, os1, os2):
    wid = lax.axis_index("s") * NC + lax.axis_index("c")
    iota = lax.iota(jnp.int32, L)

    def phase1(batch_full):
        pltpu.sync_copy(batch_hbm, batch_full)
        for grp in range(G_PAD // L):
            g = iota + grp * L

            def bs_step(_, carry):
                lo, hi = carry
                mid = lax.div(lo + hi, 2)
                probe = plsc.load_gather(batch_full, [jnp.minimum(mid, N - 1)])
                go = lo < hi
                lt = probe < g
                lo2 = jnp.where(jnp.logical_and(lt, go), mid + 1, lo)
                hi2 = jnp.where(jnp.logical_and(jnp.logical_not(lt), go),
                                mid, hi)
                return lo2, hi2

            lo0 = jnp.zeros((L,), jnp.int32)
            hi0 = jnp.full((L,), N, jnp.int32)
            lo, _ = lax.fori_loop(0, 16, bs_step, (lo0, hi0))
            starts_v[pl.ds(grp * L, L)] = lo

    pl.run_scoped(phase1, pltpu.VMEM((N,), jnp.int32))

    def phase2(x0, x1, x2, r0, r1, b0, b1, i0, i1):
        X, R, B, I = [x0, x1, x2], [r0, r1], [b0, b1], [i0, i1]
        BS, XS, GS, OS = [bs0, bs1], [xs0, xs1, xs2], [gs0, gs1], \
            [os0, os1, os2]
        tail_mask = iota >= (ROW_VECS * L - TAIL_OFF)

        def issue_b(k):
            s = k & 1
            a = (wid + k * NW) * C
            pltpu.async_copy(batch_hbm.at[pl.ds(a, C)], B[s], BS[s])

        def issue_x(k):
            s = k % 3
            a = (wid + k * NW) * C
            if k >= 3:
                pltpu.make_async_copy(
                    X[s], out_hbm.at[pl.ds(a - 3 * NW * C, C), :],
                    OS[s]).wait()
            pltpu.async_copy(x_hbm.at[pl.ds(a, C), :], X[s], XS[s])

        def stage_a(k):
            s = k & 1
            a = (wid + k * NW) * C
            pltpu.make_async_copy(batch_hbm.at[pl.ds(a, C)], B[s],
                                  BS[s]).wait()
            for j in range(C // L):
                b = B[s][pl.ds(j * L, L)]
                st = plsc.load_gather(starts_v, [b])
                pos = (a + j * L) + iota - st
                pos = jnp.minimum(jnp.maximum(pos, 0), MAX_NODES - 1)
                I[s][pl.ds(j * L, L)] = pos
            pltpu.async_copy(table_hbm.at[I[s]], R[s], GS[s])

        def stage_b(k):
            s3 = k % 3
            s = k & 1
            a = (wid + k * NW) * C
            pltpu.make_async_copy(x_hbm.at[pl.ds(a, C), :], X[s3],
                                  XS[s3]).wait()
            pltpu.make_async_copy(table_hbm.at[I[s]], R[s], GS[s]).wait()

            @plsc.parallel_loop(0, C, step=1, unroll=2)
            def _(r):
                for c in range(ROW_VECS):
                    w = pl.ds(c * L, L)
                    plsc.addupdate(X[s3].at[r, w], R[s][r, w])
                w = pl.ds(TAIL_OFF, L)
                rv = jnp.where(tail_mask, R[s][r, w], 0.0)
                X[s3][r, w] = X[s3][r, w] + rv

            pltpu.async_copy(X[s3], out_hbm.at[pl.ds(a, C), :], OS[s3])

        def guarded(k, fn):
            if 0 <= k < K_ITERS:
                @pl.when(wid + k * NW < NCHUNK)
                def _():
                    fn(k)

        guarded(0, issue_b)
        guarded(1, issue_b)
        guarded(0, issue_x)
        guarded(1, issue_x)
        for k in range(K_ITERS + 1):
            guarded(k, stage_a)
            guarded(k - 1, stage_b)
            if k >= 1:
                guarded(k + 1, issue_x)
            guarded(k + 2, issue_b)

        def drain_out(k):
            s3 = k % 3
            a = (wid + k * NW) * C
            pltpu.make_async_copy(X[s3], out_hbm.at[pl.ds(a, C), :],
                                  OS[s3]).wait()

        guarded(K_ITERS - 3, drain_out)
        guarded(K_ITERS - 2, drain_out)
        guarded(K_ITERS - 1, drain_out)

    pl.run_scoped(
        phase2,
        pltpu.VMEM((C, HIDDEN), jnp.float32),
        pltpu.VMEM((C, HIDDEN), jnp.float32),
        pltpu.VMEM((C, HIDDEN), jnp.float32),
        pltpu.VMEM((C, H_PAD), jnp.float32),
        pltpu.VMEM((C, H_PAD), jnp.float32),
        pltpu.VMEM((C,), jnp.int32),
        pltpu.VMEM((C,), jnp.int32),
        pltpu.VMEM((C,), jnp.int32),
        pltpu.VMEM((C,), jnp.int32),
    )


@jax.jit
def kernel(x, batch, pos_embedding):
    mesh = plsc.VectorSubcoreMesh(core_axis_name="c", subcore_axis_name="s",
                                  num_cores=NC, num_subcores=NS)
    f = pl.kernel(
        _sc_body,
        out_type=jax.ShapeDtypeStruct((N, HIDDEN), jnp.float32),
        mesh=mesh,
        scratch_types=[
            pltpu.VMEM((G_PAD,), jnp.int32),
            pltpu.SemaphoreType.DMA,
            pltpu.SemaphoreType.DMA,
            pltpu.SemaphoreType.DMA,
            pltpu.SemaphoreType.DMA,
            pltpu.SemaphoreType.DMA,
            pltpu.SemaphoreType.DMA,
            pltpu.SemaphoreType.DMA,
            pltpu.SemaphoreType.DMA,
            pltpu.SemaphoreType.DMA,
            pltpu.SemaphoreType.DMA,
        ],
        compiler_params=pltpu.CompilerParams(needs_layout_passes=False,
                                             use_tc_tiling_on_sc=True),
    )
    table = jnp.pad(pos_embedding, ((0, 0), (0, H_PAD - HIDDEN)))
    return f(x, batch, table)


# speedup vs baseline: 4.6129x; 1.0042x over previous
import jax
import jax.numpy as jnp
from jax import lax
from jax.experimental import pallas as pl
from jax.experimental.pallas import tpu as pltpu
from jax.experimental.pallas import tpu_sc as plsc

N = 50000
HIDDEN = 250
MAX_NODES = 1000
NUM_GRAPHS = 100

NC = 2
NS = 16
L = 16
NW = NC * NS

C = 80
NCHUNK = N // C
K_ITERS = (NCHUNK + NW - 1) // NW
G_PAD = 112
ROW_VECS = HIDDEN // L
TAIL_OFF = HIDDEN - L
H_PAD = 256


def _sc_body(x_hbm, batch_hbm, table_hbm, out_hbm, starts_v,
             b0, b1, i0, i1,
             bs0, bs1, xs0, xs1, xs2, gs0, gs1, os0, os1, os2):
    wid = lax.axis_index("s") * NC + lax.axis_index("c")
    iota = lax.iota(jnp.int32, L)
    B, I = [b0, b1], [i0, i1]
    BS = [bs0, bs1]

    def issue_b(k):
        s = k & 1
        a = (wid + k * NW) * C
        pltpu.async_copy(batch_hbm.at[pl.ds(a, C)], B[s], BS[s])

    def guarded(k, fn):
        if 0 <= k < K_ITERS:
            @pl.when(wid + k * NW < NCHUNK)
            def _():
                fn(k)

    guarded(0, issue_b)
    guarded(1, issue_b)

    def phase1(batch_full):
        pltpu.sync_copy(batch_hbm, batch_full)
        NG = G_PAD // L

        def bs_step(_, carry):
            los, his = carry
            los2, his2 = [], []
            for grp in range(NG):
                lo, hi = los[grp], his[grp]
                g = iota + grp * L
                mid = lax.div(lo + hi, 2)
                probe = plsc.load_gather(batch_full,
                                         [jnp.minimum(mid, N - 1)])
                go = lo < hi
                lt = probe < g
                los2.append(jnp.where(jnp.logical_and(lt, go), mid + 1, lo))
                his2.append(jnp.where(
                    jnp.logical_and(jnp.logical_not(lt), go), mid, hi))
            return tuple(los2), tuple(his2)

        lo0 = tuple(jnp.zeros((L,), jnp.int32) for _ in range(NG))
        hi0 = tuple(jnp.full((L,), N, jnp.int32) for _ in range(NG))
        los, _ = lax.fori_loop(0, 16, bs_step, (lo0, hi0))
        for grp in range(NG):
            starts_v[pl.ds(grp * L, L)] = los[grp]

    pl.run_scoped(phase1, pltpu.VMEM((N,), jnp.int32))

    def phase2(x0, x1, x2, r0, r1):
        X, R = [x0, x1, x2], [r0, r1]
        XS, GS, OS = [xs0, xs1, xs2], [gs0, gs1], [os0, os1, os2]
        tail_mask = iota >= (ROW_VECS * L - TAIL_OFF)

        def issue_x(k):
            s = k % 3
            a = (wid + k * NW) * C
            if k >= 3:
                pltpu.make_async_copy(
                    X[s], out_hbm.at[pl.ds(a - 3 * NW * C, C), :],
                    OS[s]).wait()
            pltpu.async_copy(x_hbm.at[pl.ds(a, C), :], X[s], XS[s])

        def stage_a(k):
            s = k & 1
            a = (wid + k * NW) * C
            pltpu.make_async_copy(batch_hbm.at[pl.ds(a, C)], B[s],
                                  BS[s]).wait()
            for j in range(C // L):
                b = B[s][pl.ds(j * L, L)]
                st = plsc.load_gather(starts_v, [b])
                pos = (a + j * L) + iota - st
                pos = jnp.minimum(jnp.maximum(pos, 0), MAX_NODES - 1)
                I[s][pl.ds(j * L, L)] = pos
            pltpu.async_copy(table_hbm.at[I[s]], R[s], GS[s])

        def stage_b(k):
            s3 = k % 3
            s = k & 1
            a = (wid + k * NW) * C
            pltpu.make_async_copy(x_hbm.at[pl.ds(a, C), :], X[s3],
                                  XS[s3]).wait()
            pltpu.make_async_copy(table_hbm.at[I[s]], R[s], GS[s]).wait()

            @plsc.parallel_loop(0, C, step=1, unroll=2)
            def _(r):
                for c in range(ROW_VECS):
                    w = pl.ds(c * L, L)
                    plsc.addupdate(X[s3].at[r, w], R[s][r, w])
                w = pl.ds(TAIL_OFF, L)
                rv = jnp.where(tail_mask, R[s][r, w], 0.0)
                X[s3][r, w] = X[s3][r, w] + rv

            pltpu.async_copy(X[s3], out_hbm.at[pl.ds(a, C), :], OS[s3])

        def guarded(k, fn):
            if 0 <= k < K_ITERS:
                @pl.when(wid + k * NW < NCHUNK)
                def _():
                    fn(k)

        guarded(0, issue_x)
        guarded(1, issue_x)
        for k in range(K_ITERS + 1):
            guarded(k, stage_a)
            guarded(k - 1, stage_b)
            if k >= 1:
                guarded(k + 1, issue_x)
            guarded(k + 2, issue_b)

        def drain_out(k):
            s3 = k % 3
            a = (wid + k * NW) * C
            pltpu.make_async_copy(X[s3], out_hbm.at[pl.ds(a, C), :],
                                  OS[s3]).wait()

        guarded(K_ITERS - 3, drain_out)
        guarded(K_ITERS - 2, drain_out)
        guarded(K_ITERS - 1, drain_out)

    pl.run_scoped(
        phase2,
        pltpu.VMEM((C, HIDDEN), jnp.float32),
        pltpu.VMEM((C, HIDDEN), jnp.float32),
        pltpu.VMEM((C, HIDDEN), jnp.float32),
        pltpu.VMEM((C, H_PAD), jnp.float32),
        pltpu.VMEM((C, H_PAD), jnp.float32),
    )


@jax.jit
def kernel(x, batch, pos_embedding):
    mesh = plsc.VectorSubcoreMesh(core_axis_name="c", subcore_axis_name="s",
                                  num_cores=NC, num_subcores=NS)
    f = pl.kernel(
        _sc_body,
        out_type=jax.ShapeDtypeStruct((N, HIDDEN), jnp.float32),
        mesh=mesh,
        scratch_types=[
            pltpu.VMEM((G_PAD,), jnp.int32),
            pltpu.VMEM((C,), jnp.int32),
            pltpu.VMEM((C,), jnp.int32),
            pltpu.VMEM((C,), jnp.int32),
            pltpu.VMEM((C,), jnp.int32),
            pltpu.SemaphoreType.DMA,
            pltpu.SemaphoreType.DMA,
            pltpu.SemaphoreType.DMA,
            pltpu.SemaphoreType.DMA,
            pltpu.SemaphoreType.DMA,
            pltpu.SemaphoreType.DMA,
            pltpu.SemaphoreType.DMA,
            pltpu.SemaphoreType.DMA,
            pltpu.SemaphoreType.DMA,
            pltpu.SemaphoreType.DMA,
        ],
        compiler_params=pltpu.CompilerParams(needs_layout_passes=False,
                                             use_tc_tiling_on_sc=True),
    )
    table = jnp.pad(pos_embedding, ((0, 0), (0, H_PAD - HIDDEN)))
    return f(x, batch, table)


# speedup vs baseline: 4.9387x; 1.0706x over previous
import jax
import jax.numpy as jnp
from jax import lax
from jax.experimental import pallas as pl
from jax.experimental.pallas import tpu as pltpu
from jax.experimental.pallas import tpu_sc as plsc

N = 50000
HIDDEN = 250
MAX_NODES = 1000
NUM_GRAPHS = 100

NC = 2
NS = 16
L = 16
NW = NC * NS

C = 80
NCHUNK = N // C
K_ITERS = (NCHUNK + NW - 1) // NW
G_PAD = 112
ROW_VECS = HIDDEN // L
TAIL_OFF = HIDDEN - L
H_PAD = 256


def _sc_body(x_hbm, batch_hbm, table_hbm, out_hbm, starts_v,
             b0, b1, i0, i1,
             bs0, bs1, xs0, xs1, xs2, gs0, gs1, os0, os1, os2):
    wid = lax.axis_index("s") * NC + lax.axis_index("c")
    iota = lax.iota(jnp.int32, L)
    B, I = [b0, b1], [i0, i1]
    BS = [bs0, bs1]

    def issue_b(k):
        s = k & 1
        a = (wid + k * NW) * C
        pltpu.async_copy(batch_hbm.at[pl.ds(a, C)], B[s], BS[s])

    def guarded(k, fn):
        if 0 <= k < K_ITERS:
            @pl.when(wid + k * NW < NCHUNK)
            def _():
                fn(k)

    guarded(0, issue_b)
    guarded(1, issue_b)

    def phase1(batch_full):
        pltpu.sync_copy(batch_hbm, batch_full)
        NG = G_PAD // L

        def bs_step(_, carry):
            los, his = carry
            los2, his2 = [], []
            for grp in range(NG):
                lo, hi = los[grp], his[grp]
                g = iota + grp * L
                mid = lax.div(lo + hi, 2)
                probe = plsc.load_gather(batch_full,
                                         [jnp.minimum(mid, N - 1)])
                go = lo < hi
                lt = probe < g
                los2.append(jnp.where(jnp.logical_and(lt, go), mid + 1, lo))
                his2.append(jnp.where(
                    jnp.logical_and(jnp.logical_not(lt), go), mid, hi))
            return tuple(los2), tuple(his2)

        lo0 = tuple(jnp.zeros((L,), jnp.int32) for _ in range(NG))
        hi0 = tuple(jnp.full((L,), N, jnp.int32) for _ in range(NG))
        los, _ = lax.fori_loop(0, 16, bs_step, (lo0, hi0))
        for grp in range(NG):
            starts_v[pl.ds(grp * L, L)] = los[grp]

    pl.run_scoped(phase1, pltpu.VMEM((N,), jnp.int32))

    def phase2(x0, x1, x2, r0, r1):
        X, R = [x0, x1, x2], [r0, r1]
        XS, GS, OS = [xs0, xs1, xs2], [gs0, gs1], [os0, os1, os2]
        tail_mask = iota >= (ROW_VECS * L - TAIL_OFF)

        def valid(k):
            chunk = wid + k * NW
            return jnp.logical_and(chunk >= 0, chunk < NCHUNK)

        def issue_x_first(k):
            a = (wid + k * NW) * C
            pltpu.async_copy(x_hbm.at[pl.ds(a, C), :], X[k % 3], XS[k % 3])

        def iteration(k, j):
            sa = j & 1

            @pl.when(valid(k))
            def _():
                a = (wid + k * NW) * C
                pltpu.make_async_copy(batch_hbm.at[pl.ds(a, C)], B[sa],
                                      BS[sa]).wait()
                for jj in range(C // L):
                    b = B[sa][pl.ds(jj * L, L)]
                    st = plsc.load_gather(starts_v, [b])
                    pos = (a + jj * L) + iota - st
                    pos = jnp.minimum(jnp.maximum(pos, 0), MAX_NODES - 1)
                    I[sa][pl.ds(jj * L, L)] = pos
                pltpu.async_copy(table_hbm.at[I[sa]], R[sa], GS[sa])

            s3 = (j - 1) % 3
            sb = (j - 1) % 2

            @pl.when(valid(k - 1))
            def _():
                a = (wid + (k - 1) * NW) * C
                pltpu.make_async_copy(x_hbm.at[pl.ds(a, C), :], X[s3],
                                      XS[s3]).wait()
                pltpu.make_async_copy(table_hbm.at[I[sb]], R[sb],
                                      GS[sb]).wait()

                @plsc.parallel_loop(0, C, step=1, unroll=2)
                def _(r):
                    for c in range(ROW_VECS):
                        w = pl.ds(c * L, L)
                        plsc.addupdate(X[s3].at[r, w], R[sb][r, w])
                    w = pl.ds(TAIL_OFF, L)
                    rv = jnp.where(tail_mask, R[sb][r, w], 0.0)
                    X[s3][r, w] = X[s3][r, w] + rv

                pltpu.async_copy(X[s3], out_hbm.at[pl.ds(a, C), :],
                                 OS[s3])

            sx = (j + 1) % 3

            @pl.when(jnp.logical_and(k >= 1, valid(k + 1)))
            def _():
                a = (wid + (k + 1) * NW) * C

                @pl.when(k >= 2)
                def _():
                    pltpu.make_async_copy(
                        X[sx], out_hbm.at[pl.ds(a - 3 * NW * C, C), :],
                        OS[sx]).wait()

                pltpu.async_copy(x_hbm.at[pl.ds(a, C), :], X[sx], XS[sx])

            @pl.when(valid(k + 2))
            def _():
                a = (wid + (k + 2) * NW) * C
                pltpu.async_copy(batch_hbm.at[pl.ds(a, C)], B[sa], BS[sa])

        @pl.when(valid(0))
        def _():
            issue_x_first(0)

        @pl.when(valid(1))
        def _():
            issue_x_first(1)

        def six_iters(m, _):
            for j in range(6):
                iteration(m * 6 + j, j)
            return 0

        lax.fori_loop(0, (K_ITERS + 1 + 5) // 6, six_iters, 0)

        def drain_out(k):
            if not 0 <= k < K_ITERS:
                return
            s3 = k % 3
            a = (wid + k * NW) * C

            @pl.when(valid(k))
            def _():
                pltpu.make_async_copy(X[s3], out_hbm.at[pl.ds(a, C), :],
                                      OS[s3]).wait()

        drain_out(K_ITERS - 3)
        drain_out(K_ITERS - 2)
        drain_out(K_ITERS - 1)

    pl.run_scoped(
        phase2,
        pltpu.VMEM((C, HIDDEN), jnp.float32),
        pltpu.VMEM((C, HIDDEN), jnp.float32),
        pltpu.VMEM((C, HIDDEN), jnp.float32),
        pltpu.VMEM((C, H_PAD), jnp.float32),
        pltpu.VMEM((C, H_PAD), jnp.float32),
    )


@jax.jit
def kernel(x, batch, pos_embedding):
    mesh = plsc.VectorSubcoreMesh(core_axis_name="c", subcore_axis_name="s",
                                  num_cores=NC, num_subcores=NS)
    f = pl.kernel(
        _sc_body,
        out_type=jax.ShapeDtypeStruct((N, HIDDEN), jnp.float32),
        mesh=mesh,
        scratch_types=[
            pltpu.VMEM((G_PAD,), jnp.int32),
            pltpu.VMEM((C,), jnp.int32),
            pltpu.VMEM((C,), jnp.int32),
            pltpu.VMEM((C,), jnp.int32),
            pltpu.VMEM((C,), jnp.int32),
            pltpu.SemaphoreType.DMA,
            pltpu.SemaphoreType.DMA,
            pltpu.SemaphoreType.DMA,
            pltpu.SemaphoreType.DMA,
            pltpu.SemaphoreType.DMA,
            pltpu.SemaphoreType.DMA,
            pltpu.SemaphoreType.DMA,
            pltpu.SemaphoreType.DMA,
            pltpu.SemaphoreType.DMA,
            pltpu.SemaphoreType.DMA,
        ],
        compiler_params=pltpu.CompilerParams(needs_layout_passes=False,
                                             use_tc_tiling_on_sc=True),
    )
    table = jnp.pad(pos_embedding, ((0, 0), (0, H_PAD - HIDDEN)))
    return f(x, batch, table)
